# baseline (speedup 1.0000x reference)
import functools

import jax
import jax.numpy as jnp
from jax import lax
from jax.experimental import pallas as pl
from jax.experimental.pallas import tpu as pltpu
from jax.experimental.pallas import tpu_sc as plsc

_PAD_ROW = 5
_NK = 6
_D = 128
_NC = 2
_NS = 16
_NW = _NC * _NS
_CHUNK = 128
_L = 16


def _body(idx_hbm, table_hbm, out_hbm, idx_v, table_v, rows_v, osem):
    c = lax.axis_index("c")
    s = lax.axis_index("s")
    wid = s * _NC + c
    n_chunks = idx_hbm.shape[0] // _NW
    base = wid * n_chunks

    pltpu.sync_copy(idx_hbm.at[pl.ds(base, n_chunks)], idx_v)
    pltpu.sync_copy(table_hbm, table_v)
    zero = jnp.zeros((_L,), jnp.float32)
    for jb in range(_D // _L):
        table_v[pl.ds(_PAD_ROW * _D + jb * _L, _L)] = zero

    cw = _CHUNK * _D

    def start_out(i, slot):
        pltpu.make_async_copy(
            rows_v.at[pl.ds(slot * cw, cw)],
            out_hbm.at[pl.ds((base + i) * cw, cw)], osem).start()

    def wait_out():
        pltpu.make_async_copy(
            rows_v.at[pl.ds(0, cw)], out_hbm.at[pl.ds(0, cw)], osem).wait()

    def chunk(i, carry):
        slot = lax.rem(i, 2)

        @pl.when(i >= 2)
        def _():
            wait_out()

        def group(g, carry2):
            ids = idx_v[i, pl.ds(g * _L, _L)]
            gbase = (slot * _CHUNK + g * _L) * _D
            koffs = [lax.shift_left(ids[l], 7) for l in range(_L)]

            def load_row(l):
                return [table_v[pl.ds(koffs[l] + jb * _L, _L)]
                        for jb in range(_D // _L)]

            def store_row(l, vals):
                for jb in range(_D // _L):
                    rows_v[pl.ds(gbase + l * _D + jb * _L, _L)] = vals[jb]

            vals = load_row(0)
            for l in range(1, _L):
                nxt = load_row(l)
                store_row(l - 1, vals)
                vals = nxt
            store_row(_L - 1, vals)
            return carry2

        lax.fori_loop(0, _CHUNK // _L, group, 0)
        start_out(i, slot)
        return carry

    lax.fori_loop(0, n_chunks, chunk, 0)
    wait_out()
    wait_out()


def kernel(inputs, table):
    n_rows, n_cols = inputs.shape
    b_total = n_rows * n_cols
    idx = inputs.reshape(b_total // _CHUNK, _CHUNK).astype(jnp.int32)

    mesh = plsc.VectorSubcoreMesh(core_axis_name="c", subcore_axis_name="s")

    run = functools.partial(
        pl.kernel,
        out_type=jax.ShapeDtypeStruct((b_total * _D,), jnp.float32),
        mesh=mesh,
        scratch_types=[
            pltpu.VMEM((b_total // _CHUNK // _NW, _CHUNK), jnp.int32),
            pltpu.VMEM((_NK * _D,), jnp.float32),
            pltpu.VMEM((2 * _CHUNK * _D,), jnp.float32),
            pltpu.SemaphoreType.DMA,
        ],
    )(_body)

    out = run(idx, table.reshape(_NK * _D))
    return out.reshape(n_rows, n_cols, _D)


# speedup vs baseline: 1.0016x; 1.0016x over previous
import functools

import jax
import jax.numpy as jnp
from jax import lax
from jax.experimental import pallas as pl
from jax.experimental.pallas import tpu as pltpu
from jax.experimental.pallas import tpu_sc as plsc

_PAD_ROW = 5
_NK = 6
_D = 128
_NC = 2
_NS = 16
_NW = _NC * _NS
_SW = 128
_CPC = 2
_CHUNK = _SW * _CPC
_L = 16


def _body(idx_hbm, table_hbm, out_hbm, idx_v, table_v, rows_v, osem):
    c = lax.axis_index("c")
    s = lax.axis_index("s")
    wid = s * _NC + c
    n_stage = idx_hbm.shape[0] // _NW
    n_chunks = n_stage // _CPC
    base = wid * n_stage

    pltpu.sync_copy(idx_hbm.at[pl.ds(base, n_stage)], idx_v)
    pltpu.sync_copy(table_hbm, table_v)
    zero = jnp.zeros((_L,), jnp.float32)
    for jb in range(_D // _L):
        table_v[pl.ds(_PAD_ROW * _D + jb * _L, _L)] = zero

    cw = _CHUNK * _D

    def start_out(i, slot):
        pltpu.make_async_copy(
            rows_v.at[pl.ds(slot * cw, cw)],
            out_hbm.at[pl.ds(base * (_SW * _D) + i * cw, cw)], osem).start()

    def wait_out():
        pltpu.make_async_copy(
            rows_v.at[pl.ds(0, cw)], out_hbm.at[pl.ds(0, cw)], osem).wait()

    def chunk(i, carry):
        slot = lax.rem(i, 2)

        @pl.when(i >= 2)
        def _():
            wait_out()

        for sr in range(_CPC):
            def group(g, carry2, sr=sr):
                ids = idx_v[i * _CPC + sr, pl.ds(g * _L, _L)]
                gbase = (slot * _CHUNK + sr * _SW + g * _L) * _D
                koffs = [lax.shift_left(ids[l], 7) for l in range(_L)]

                def load_row(l):
                    return [table_v[pl.ds(koffs[l] + jb * _L, _L)]
                            for jb in range(_D // _L)]

                def store_row(l, vals):
                    for jb in range(_D // _L):
                        rows_v[pl.ds(gbase + l * _D + jb * _L, _L)] = vals[jb]

                vals = load_row(0)
                for l in range(1, _L):
                    nxt = load_row(l)
                    store_row(l - 1, vals)
                    vals = nxt
                store_row(_L - 1, vals)
                return carry2

            lax.fori_loop(0, _SW // _L, group, 0)
        start_out(i, slot)
        return carry

    lax.fori_loop(0, n_chunks, chunk, 0)
    wait_out()
    wait_out()


def kernel(inputs, table):
    n_rows, n_cols = inputs.shape
    b_total = n_rows * n_cols
    idx = inputs.reshape(b_total // _SW, _SW).astype(jnp.int32)

    mesh = plsc.VectorSubcoreMesh(core_axis_name="c", subcore_axis_name="s")

    run = functools.partial(
        pl.kernel,
        out_type=jax.ShapeDtypeStruct((b_total * _D,), jnp.float32),
        mesh=mesh,
        scratch_types=[
            pltpu.VMEM((b_total // _SW // _NW, _SW), jnp.int32),
            pltpu.VMEM((_NK * _D,), jnp.float32),
            pltpu.VMEM((2 * _CHUNK * _D,), jnp.float32),
            pltpu.SemaphoreType.DMA,
        ],
    )(_body)

    out = run(idx, table.reshape(_NK * _D))
    return out.reshape(n_rows, n_cols, _D)
